# baseline (speedup 1.0000x reference)
import functools

import jax
import jax.numpy as jnp
from jax import lax
from jax.experimental import pallas as pl
from jax.experimental.pallas import tpu as pltpu
from jax.experimental.pallas import tpu_sc as plsc

MAXLEN = 8192
DM = 768
BATCH = 4
HALF = MAXLEN // 2
SC_BATCH = 2
TC_BATCH = BATCH - SC_BATCH
NC = 2
NS = 16
NW = NC * NS
K = HALF // NW
L = 16

_mesh = plsc.VectorSubcoreMesh(core_axis_name="c", subcore_axis_name="s")


@functools.partial(
    pl.kernel,
    mesh=_mesh,
    out_type=jax.ShapeDtypeStruct((SC_BATCH * MAXLEN, DM), jnp.float32),
    scratch_types=[
        pltpu.VMEM((K, DM), jnp.float32),
        pltpu.VMEM((SC_BATCH, K), jnp.int32),
        pltpu.VMEM((1, DM), jnp.float32),
        pltpu.SemaphoreType.DMA,
    ],
)
def _emb_sc(table_hbm, out_hbm, rows_v, idx_v, row0_v, sem):
    wid = lax.axis_index("s") * NC + lax.axis_index("c")
    s = wid * K

    pltpu.sync_copy(table_hbm.at[pl.ds(s, K)], rows_v)

    lane = lax.iota(jnp.int32, L)
    for b in range(SC_BATCH):
        base = (b * MAXLEN + HALF) - s
        for j in range(K // L):
            idx_v[b, pl.ds(j * L, L)] = (base - j * L) - lane

    copies = []
    for b in range(SC_BATCH):
        copies.append(
            pltpu.async_copy(
                rows_v, out_hbm.at[pl.ds(b * MAXLEN + HALF + s, K)], sem
            )
        )
        copies.append(pltpu.async_copy(rows_v, out_hbm.at[idx_v.at[b]], sem))
    for c in copies:
        c.wait()

    @pl.when(wid == 0)
    def _():
        pltpu.sync_copy(table_hbm.at[pl.ds(HALF, 1)], row0_v)
        for b in range(SC_BATCH):
            pltpu.sync_copy(row0_v, out_hbm.at[pl.ds(b * MAXLEN, 1)])


TC_R = 512
TC_HB = HALF // TC_R


def _tc_body(a_ref, b0_ref, b1_ref, o_ref):
    j = pl.program_id(1)

    @pl.when(j >= TC_HB)
    def _():
        o_ref[0] = a_ref[...]

    @pl.when(j < TC_HB)
    def _():
        o_ref[0, 0:1] = b1_ref[0:1]
        o_ref[0, 1:TC_R] = jnp.flip(b0_ref[1:TC_R], axis=0)


_emb_tc = pl.pallas_call(
    _tc_body,
    grid=(TC_BATCH, MAXLEN // TC_R),
    in_specs=[
        pl.BlockSpec((TC_R, DM), lambda b, j: (jnp.maximum(j - TC_HB, 0), 0)),
        pl.BlockSpec((TC_R, DM), lambda b, j: (jnp.clip(TC_HB - 1 - j, 0, TC_HB), 0)),
        pl.BlockSpec((TC_R, DM), lambda b, j: (jnp.clip(TC_HB - j, 0, TC_HB), 0)),
    ],
    out_specs=pl.BlockSpec((1, TC_R, DM), lambda b, j: (b, j, 0)),
    out_shape=jax.ShapeDtypeStruct((TC_BATCH, MAXLEN, DM), jnp.float32),
)


def kernel(x, table):
    del x
    out_a = _emb_sc(table).reshape(SC_BATCH, MAXLEN, DM)
    out_b = _emb_sc(table).reshape(SC_BATCH, MAXLEN, DM)
    return jnp.concatenate([out_a, out_b], axis=0)


# speedup vs baseline: 1.3796x; 1.3796x over previous
import functools

import jax
import jax.numpy as jnp
from jax import lax
from jax.experimental import pallas as pl
from jax.experimental.pallas import tpu as pltpu
from jax.experimental.pallas import tpu_sc as plsc

MAXLEN = 8192
DM = 768
BATCH = 4
HALF = MAXLEN // 2
NC = 2
NS = 16
NW = NC * NS
K = HALF // NW
L = 16

_mesh = plsc.VectorSubcoreMesh(core_axis_name="c", subcore_axis_name="s")


@functools.partial(
    pl.kernel,
    mesh=_mesh,
    out_type=jax.ShapeDtypeStruct((BATCH * MAXLEN, DM), jnp.float32),
    scratch_types=[
        pltpu.VMEM((K, DM), jnp.float32),
        pltpu.VMEM((BATCH, K), jnp.int32),
        pltpu.VMEM((1, DM), jnp.float32),
        pltpu.SemaphoreType.DMA,
    ],
)
def _emb_sc_mirror(table_hbm, out_hbm, rows_v, idx_v, row0_v, sem):
    wid = lax.axis_index("s") * NC + lax.axis_index("c")
    s = wid * K

    pltpu.sync_copy(table_hbm.at[pl.ds(s, K)], rows_v)

    lane = lax.iota(jnp.int32, L)
    for b in range(BATCH):
        base = (b * MAXLEN + HALF) - s
        for j in range(K // L):
            idx_v[b, pl.ds(j * L, L)] = (base - j * L) - lane

    copies = [
        pltpu.async_copy(rows_v, out_hbm.at[idx_v.at[b]], sem)
        for b in range(BATCH)
    ]
    for c in copies:
        c.wait()

    @pl.when(wid == 0)
    def _():
        pltpu.sync_copy(table_hbm.at[pl.ds(HALF, 1)], row0_v)
        for b in range(BATCH):
            pltpu.sync_copy(row0_v, out_hbm.at[pl.ds(b * MAXLEN, 1)])


TC_R = 512
TC_NJ = HALF // TC_R


def _tc_body(t_ref, buf_ref, o_ref):
    del buf_ref
    o_ref[0] = t_ref[...]


_emb_tc_fwd = pl.pallas_call(
    _tc_body,
    grid=(TC_NJ, BATCH),
    in_specs=[
        pl.BlockSpec((TC_R, DM), lambda j, b: (j, 0)),
        pl.BlockSpec(memory_space=pl.ANY),
    ],
    out_specs=pl.BlockSpec((1, TC_R, DM), lambda j, b: (b, TC_NJ + j, 0)),
    out_shape=jax.ShapeDtypeStruct((BATCH, MAXLEN, DM), jnp.float32),
    input_output_aliases={1: 0},
)


def kernel(x, table):
    del x
    buf = _emb_sc_mirror(table).reshape(BATCH, MAXLEN, DM)
    return _emb_tc_fwd(table, buf)


# speedup vs baseline: 1.8333x; 1.3288x over previous
import functools

import jax
import jax.numpy as jnp
from jax import lax
from jax.experimental import pallas as pl
from jax.experimental.pallas import tpu as pltpu
from jax.experimental.pallas import tpu_sc as plsc

MAXLEN = 8192
DM = 768
BATCH = 4
HALF = MAXLEN // 2
NC = 2
NS = 16
NW = NC * NS
K = HALF // NW
L = 16

_mesh = plsc.VectorSubcoreMesh(core_axis_name="c", subcore_axis_name="s")


@functools.partial(
    pl.kernel,
    mesh=_mesh,
    out_type=jax.ShapeDtypeStruct((BATCH * MAXLEN, DM), jnp.float32),
    scratch_types=[
        pltpu.VMEM((K, DM), jnp.float32),
        pltpu.VMEM((BATCH, K), jnp.int32),
        pltpu.VMEM((1, DM), jnp.float32),
        pltpu.SemaphoreType.DMA,
    ],
)
def _emb(table_hbm, out_hbm, rows_v, idx_v, row0_v, sem):
    wid = lax.axis_index("s") * NC + lax.axis_index("c")
    s = wid * K

    pltpu.sync_copy(table_hbm.at[pl.ds(s, K)], rows_v)

    lane = lax.iota(jnp.int32, L)
    for b in range(BATCH):
        base = (b * MAXLEN + HALF) - s
        for j in range(K // L):
            idx_v[b, pl.ds(j * L, L)] = (base - j * L) - lane

    copies = []
    for b in range(BATCH):
        copies.append(
            pltpu.async_copy(
                rows_v, out_hbm.at[pl.ds(b * MAXLEN + HALF + s, K)], sem
            )
        )
        copies.append(pltpu.async_copy(rows_v, out_hbm.at[idx_v.at[b]], sem))
    for c in copies:
        c.wait()

    @pl.when(wid == 0)
    def _():
        pltpu.sync_copy(table_hbm.at[pl.ds(HALF, 1)], row0_v)
        for b in range(BATCH):
            pltpu.sync_copy(row0_v, out_hbm.at[pl.ds(b * MAXLEN, 1)])


def kernel(x, table):
    del x
    return _emb(table).reshape(BATCH, MAXLEN, DM)
